# baseline (speedup 1.0000x reference)
import functools

import jax
import jax.numpy as jnp
from jax.experimental import pallas as pl

N = 50000
NUM_KMERS = 65536
EMB_DIM = 64
HID = 128
HEADS = 4
G = 64


def _mlp_body(comb_ref, wf1_ref, bf1_ref, wf2_ref, bf2_ref, out_ref):
    h = jnp.maximum(
        jnp.dot(comb_ref[...], wf1_ref[...], preferred_element_type=jnp.float32)
        + bf1_ref[...][None, :], 0.0)
    out_ref[...] = (
        jnp.dot(h, wf2_ref[...], preferred_element_type=jnp.float32)
        + bf2_ref[...][None, :])


def _mlp_head(comb, Wf1, bf1, Wf2, bf2):
    return pl.pallas_call(
        _mlp_body,
        out_shape=jax.ShapeDtypeStruct((comb.shape[0], Wf2.shape[1]), jnp.float32),
    )(comb, Wf1, bf1, Wf2, bf2)


def _gat_conv(h_in, src, dst, W, a_src, a_dst, bias, heads, out_ch, concat):
    n = h_in.shape[0]
    h = (h_in @ W).reshape(n, heads, out_ch)
    al = (h * a_src[None, :, :]).sum(-1)
    ar = (h * a_dst[None, :, :]).sum(-1)
    alpha = jax.nn.leaky_relu(al[src] + ar[dst], negative_slope=0.2)
    amax = jax.ops.segment_max(alpha, dst, num_segments=n)
    ex = jnp.exp(alpha - amax[dst])
    den = jax.ops.segment_sum(ex, dst, num_segments=n)
    att = ex / (den[dst] + 1e-16)
    out = jax.ops.segment_sum(h[src] * att[:, :, None], dst, num_segments=n)
    if concat:
        out = out.reshape(n, heads * out_ch)
    else:
        out = out.mean(axis=1)
    return out + bias


def kernel(x, edge_index, batch, card, genome_feat, emb, W1, a_src1, a_dst1, b1,
           W2, a_src2, a_dst2, b2, Wc1, bc1, Wc2, bc2, Wg, bg, Wf1, bf1, Wf2, bf2):
    h = jnp.take(emb, x[:, 0], axis=0)
    n = h.shape[0]
    loop = jnp.arange(n)
    src = jnp.concatenate([edge_index[0], loop])
    dst = jnp.concatenate([edge_index[1], loop])
    h = jax.nn.elu(_gat_conv(h, src, dst, W1, a_src1, a_dst1, b1, HEADS, HID // HEADS, True))
    h = jax.nn.elu(_gat_conv(h, src, dst, W2, a_src2, a_dst2, b2, 1, HID, False))
    cnt = jax.ops.segment_sum(jnp.ones((n, 1), dtype=h.dtype), batch, num_segments=G)
    gmean = jax.ops.segment_sum(h, batch, num_segments=G) / jnp.maximum(cnt, 1.0)
    gmax = jax.ops.segment_max(h, batch, num_segments=G)
    xg = jnp.concatenate([gmean, gmax], axis=1)
    c = jax.nn.relu(card @ Wc1 + bc1)
    c = jax.nn.relu(c @ Wc2 + bc2)
    g = jax.nn.relu(genome_feat @ Wg + bg)
    comb = jnp.concatenate([xg, c, g], axis=1)
    return _mlp_head(comb, Wf1, bf1, Wf2, bf2)


# speedup vs baseline: 2.5657x; 2.5657x over previous
import functools

import jax
import jax.numpy as jnp
from jax import lax
from jax.experimental import pallas as pl
from jax.experimental.pallas import tpu as pltpu
from jax.experimental.pallas import tpu_sc as plsc

N = 50000
NUM_KMERS = 65536
EMB_DIM = 64
HID = 128
HEADS = 4
G = 64

N_PAD = 50176
E_TOT = N + 800000
E_PAD = 851968
ROW = 36
ARW = 16
NB_NODE = 14
BSZ = 64
CHUNK = 1568
CAP = 1280
EW = E_PAD // 32
NB_EDGE = (32 * CAP) // BSZ
EPT = 53248
NPS = N_PAD // 16
BLK = 512
f32 = jnp.float32
i32 = jnp.int32


def _ktab_body(emb_ref, w1_ref, asrc_ref, adst_ref, r0, r1, r2, r3, arr):
    t = jnp.dot(emb_ref[...], w1_ref[...], preferred_element_type=f32)
    al = jnp.dot(t, asrc_ref[...], preferred_element_type=f32)
    ar = jnp.dot(t, adst_ref[...], preferred_element_type=f32)
    z7 = jnp.zeros((BLK, 3), f32)
    for h, ref in enumerate((r0, r1, r2, r3)):
        ref[...] = jnp.concatenate(
            [t[:, 32 * h:32 * h + 32], al[:, h:h + 1], z7], axis=1)
    arr[...] = jnp.concatenate([ar, jnp.zeros((BLK, 12), f32)], axis=1)


def _build_kmer_tables(emb, W1, Asrc, Adst):
    nblk = NUM_KMERS // BLK
    rec_t = jax.ShapeDtypeStruct((NUM_KMERS, ROW), f32)
    return pl.pallas_call(
        _ktab_body,
        grid=(nblk,),
        in_specs=[
            pl.BlockSpec((BLK, EMB_DIM), lambda i: (i, 0)),
            pl.BlockSpec((EMB_DIM, HID), lambda i: (0, 0)),
            pl.BlockSpec((HID, HEADS), lambda i: (0, 0)),
            pl.BlockSpec((HID, HEADS), lambda i: (0, 0)),
        ],
        out_specs=[pl.BlockSpec((BLK, ROW), lambda i: (i, 0))] * 4
        + [pl.BlockSpec((BLK, ARW), lambda i: (i, 0))],
        out_shape=[rec_t] * 4 + [jax.ShapeDtypeStruct((NUM_KMERS, ARW), f32)],
    )(emb, W1, Asrc, Adst)


def _gather_nodes(x_pad, rk0, rk1, rk2, rk3, ark):
    mesh = plsc.VectorSubcoreMesh(core_axis_name="c", subcore_axis_name="s")

    @functools.partial(
        pl.kernel,
        out_type=(jax.ShapeDtypeStruct((HEADS, N_PAD, ROW), f32),
                  jax.ShapeDtypeStruct((N_PAD, ARW), f32)),
        mesh=mesh,
        compiler_params=pltpu.CompilerParams(use_tc_tiling_on_sc=False,
                                             needs_layout_passes=False),
        scratch_types=[
            pltpu.VMEM((112,), i32),
            pltpu.VMEM((112, ROW), f32),
            pltpu.VMEM((112, ARW), f32),
            pltpu.SemaphoreType.DMA,
        ],
    )
    def k(xp_h, k0, k1, k2, k3, ka, orec, oar, idx_v, b40, b16, sem):
        w = lax.axis_index("s") * 2 + lax.axis_index("c")
        base = w * (N_PAD // 32)

        def bloop(b, carry):
            nb = base + b * 112
            pltpu.sync_copy(xp_h.at[pl.ds(nb, 112)], idx_v)
            for h, kt in enumerate((k0, k1, k2, k3)):
                pltpu.async_copy(kt.at[idx_v], b40, sem).wait()
                pltpu.sync_copy(b40, orec.at[h, pl.ds(nb, 112)])
            pltpu.async_copy(ka.at[idx_v], b16, sem).wait()
            pltpu.sync_copy(b16, oar.at[pl.ds(nb, 112)])
            return carry

        lax.fori_loop(0, NB_NODE, bloop, 0)

    return k(x_pad, rk0, rk1, rk2, rk3, ark)


def _bucket_edges(src_p, dst_p, psrc, pdst):
    mesh = plsc.VectorSubcoreMesh(core_axis_name="c", subcore_axis_name="s")

    @functools.partial(
        pl.kernel,
        out_type=(jax.ShapeDtypeStruct((32, 32, CAP), i32),
                  jax.ShapeDtypeStruct((32, 32, CAP), i32)),
        mesh=mesh,
        compiler_params=pltpu.CompilerParams(use_tc_tiling_on_sc=False,
                                             needs_layout_passes=False),
        scratch_types=[
            pltpu.VMEM((2048,), i32),
            pltpu.VMEM((2048,), i32),
            pltpu.VMEM((32 * CAP,), i32),
            pltpu.VMEM((32 * CAP,), i32),
            pltpu.SMEM((32,), i32),
        ],
    )
    def k(src_h, dst_h, ps_h, pd_h, osrc, odst, ebs, ebd, cs, cd, cnts):
        w = lax.axis_index("s") * 2 + lax.axis_index("c")
        iota = lax.iota(i32, 16)
        pltpu.sync_copy(ps_h, cs)
        pltpu.sync_copy(pd_h, cd)
        for b in range(32):
            cnts[b] = 0

        def chunk_loop(cb, carry):
            base = w * EW + cb * 2048
            pltpu.sync_copy(src_h.at[pl.ds(base, 2048)], ebs)
            pltpu.sync_copy(dst_h.at[pl.ds(base, 2048)], ebd)

            def vec_loop(v, vcarry):
                rows = v * 16 + iota
                sv = plsc.load_gather(ebs, [rows])
                dv = plsc.load_gather(ebd, [rows])
                bv = dv // CHUNK
                for b in range(32):
                    m = bv == b
                    cnt = cnts[b]
                    plsc.store_compressed(
                        cs.at[pl.ds(b * CAP + cnt, 16)], sv, mask=m)
                    plsc.store_compressed(
                        cd.at[pl.ds(b * CAP + cnt, 16)], dv, mask=m)
                    inc = lax.reduce_max(
                        plsc.all_reduce_population_count(m), (0,))
                    cnts[b] = jnp.minimum(cnt + inc, CAP - 16)
                return vcarry

            lax.fori_loop(0, 128, vec_loop, 0)
            return carry

        lax.fori_loop(0, EW // 2048, chunk_loop, 0)
        for b in range(32):
            tail = b * CAP + cnts[b] + iota
            plsc.store_scatter(cs, [tail], iota * 0 + N)
            plsc.store_scatter(cd, [tail], iota * 0 + b * CHUNK)
        for b in range(32):
            pltpu.sync_copy(cs.at[pl.ds(b * CAP, CAP)], osrc.at[b, w])
            pltpu.sync_copy(cd.at[pl.ds(b * CAP, CAP)], odst.at[b, w])

    return k(src_p, dst_p, psrc, pdst)


def _edge_pass(lane_mult, bsrc, bdst, r0a, r1a, r2a, r3a, arn, zrows):
    mesh = plsc.VectorSubcoreMesh(core_axis_name="c", subcore_axis_name="s")

    @functools.partial(
        pl.kernel,
        out_type=jax.ShapeDtypeStruct((HEADS, N_PAD, ROW), f32),
        mesh=mesh,
        compiler_params=pltpu.CompilerParams(use_tc_tiling_on_sc=False,
                                             needs_layout_passes=False),
        scratch_types=[
            pltpu.VMEM((BSZ,), i32),
            pltpu.VMEM((BSZ,), i32),
            pltpu.VMEM((BSZ, ROW), f32),
            pltpu.VMEM((CHUNK, ARW), f32),
            pltpu.VMEM((CHUNK, ROW), f32),
            pltpu.SemaphoreType.DMA,
        ],
    )
    def k(bs_h, bd_h, rec0, rec1, rec2, rec3, ar_h, z_h, out_h,
          idxs, idxd, recb, arc, acc, sem):
        w = lax.axis_index("s") * 2 + lax.axis_index("c")
        iota = lax.iota(i32, 16)
        c32 = iota * 0 + 32
        pltpu.sync_copy(ar_h.at[pl.ds(w * CHUNK, CHUNK)], arc)

        for p, rec_t in enumerate((rec0, rec1, rec2, rec3)):
            lanev = iota * 0 + p * lane_mult
            pltpu.sync_copy(z_h.at[pl.ds(0, CHUNK)], acc)

            def bloop(b, carry):
                eb = w * (32 * CAP) + b * BSZ
                pltpu.sync_copy(bs_h.at[pl.ds(eb, BSZ)], idxs)
                pltpu.sync_copy(bd_h.at[pl.ds(eb, BSZ)], idxd)
                pltpu.async_copy(rec_t.at[idxs], recb, sem).wait()

                def gloop(g, gcarry):
                    rows = g * 16 + iota
                    ldv = plsc.load_gather(idxd, [rows]) - w * CHUNK
                    al = plsc.load_gather(recb, [rows, c32])
                    ar = plsc.load_gather(arc, [ldv, lanev])
                    t = al + ar
                    t = jnp.maximum(t, 0.0) + 0.2 * jnp.minimum(t, 0.0)
                    ex = jnp.exp(t)
                    iota16p = iota + 16
                    m0 = iota == 0
                    for e in range(16):
                        m = iota == e
                        ld_e = lax.reduce_max(jnp.where(m, ldv, 0), (0,))
                        ex_e = lax.reduce_max(jnp.where(m, ex, 0.0), (0,))
                        ldf = iota * 0 + ld_e
                        exf = (iota * 0).astype(f32) + ex_e
                        rf = iota * 0 + g * 16 + e
                        h0 = plsc.load_gather(recb, [rf, iota])
                        plsc.addupdate_scatter(acc, [ldf, iota], h0 * exf)
                        h1 = plsc.load_gather(recb, [rf, iota16p])
                        plsc.addupdate_scatter(acc, [ldf, iota16p],
                                               h1 * exf)
                        plsc.addupdate_scatter(acc, [ldf, c32], exf,
                                               mask=m0)
                    return gcarry

                lax.fori_loop(0, BSZ // 16, gloop, 0)
                return carry

            lax.fori_loop(0, NB_EDGE, bloop, 0)
            pltpu.sync_copy(acc, out_h.at[p, pl.ds(w * CHUNK, CHUNK)])

    return k(bsrc, bdst, r0a, r1a, r2a, r3a, arn, zrows)


def _mid_body(acc_ref, w2_ref, b1_ref, as2_ref, ad2_ref, rec2_ref, ar2_ref):
    parts = []
    for h in range(HEADS):
        blk = acc_ref[h]
        parts.append(blk[:, 0:32] / (blk[:, 32:33] + 1e-16))
    x = jnp.concatenate(parts, axis=1) + b1_ref[...]
    h2in = jnp.where(x > 0, x, jnp.exp(x) - 1.0)
    h2 = jnp.dot(h2in, w2_ref[...], preferred_element_type=f32)
    al2 = jnp.dot(h2, as2_ref[...].T, preferred_element_type=f32)
    ar2 = jnp.dot(h2, ad2_ref[...].T, preferred_element_type=f32)
    z7 = jnp.zeros((BLK, 3), f32)
    rec2_ref[...] = jnp.stack(
        [jnp.concatenate([h2[:, 32 * q:32 * q + 32], al2, z7], axis=1)
         for q in range(4)], axis=0)
    ar2_ref[...] = jnp.concatenate([ar2, jnp.zeros((BLK, ARW - 1), f32)],
                                   axis=1)


def _mid(acc1, W2, b1r, as2, ad2):
    nblk = N_PAD // BLK
    return pl.pallas_call(
        _mid_body,
        grid=(nblk,),
        in_specs=[
            pl.BlockSpec((HEADS, BLK, ROW), lambda i: (0, i, 0)),
            pl.BlockSpec((HID, HID), lambda i: (0, 0)),
            pl.BlockSpec((1, HID), lambda i: (0, 0)),
            pl.BlockSpec((1, HID), lambda i: (0, 0)),
            pl.BlockSpec((1, HID), lambda i: (0, 0)),
        ],
        out_specs=[
            pl.BlockSpec((HEADS, BLK, ROW), lambda i: (0, i, 0)),
            pl.BlockSpec((BLK, ARW), lambda i: (i, 0)),
        ],
        out_shape=[
            jax.ShapeDtypeStruct((HEADS, N_PAD, ROW), f32),
            jax.ShapeDtypeStruct((N_PAD, ARW), f32),
        ],
    )(acc1, W2, b1r, as2, ad2)


def _pool_body(lohi_ref, acc_ref, b2_ref, batch_ref, osum, ocnt, omax):
    i = pl.program_id(0)

    @pl.when(i == 0)
    def _():
        osum[...] = jnp.zeros_like(osum)
        ocnt[...] = jnp.zeros_like(ocnt)
        omax[...] = jnp.full_like(omax, -jnp.inf)

    parts = []
    for q in range(4):
        blk = acc_ref[q]
        parts.append(blk[:, 0:32] / (blk[:, 32:33] + 1e-16))
    x = jnp.concatenate(parts, axis=1) + b2_ref[...]
    h3 = jnp.where(x > 0, x, jnp.exp(x) - 1.0)
    bvec = batch_ref[...]
    lo = lohi_ref[0, 0, 0]
    hi = lohi_ref[0, 0, 1]

    def gbody(g, carry):
        m = bvec == g
        ms = jnp.where(m, h3, 0.0)
        osum[pl.ds(g, 1), :] = (osum[pl.ds(g, 1), :]
                                + jnp.sum(ms, axis=0, keepdims=True))
        ocnt[pl.ds(g, 1), :] = (ocnt[pl.ds(g, 1), :]
                                + jnp.sum(jnp.where(m, 1.0, 0.0), axis=0,
                                          keepdims=True))
        mm = jnp.where(m, h3, -jnp.inf)
        omax[pl.ds(g, 1), :] = jnp.maximum(
            omax[pl.ds(g, 1), :], jnp.max(mm, axis=0, keepdims=True))
        return carry

    lax.fori_loop(lo, hi + 1, gbody, 0)


def _pool(acc2, b2r, batch_col, lohi):
    nblk = N_PAD // BLK
    out_t = jax.ShapeDtypeStruct((G, HID), f32)
    return pl.pallas_call(
        _pool_body,
        grid=(nblk,),
        in_specs=[
            pl.BlockSpec((1, 1, 2), lambda i: (i, 0, 0),
                         memory_space=pltpu.SMEM),
            pl.BlockSpec((HEADS, BLK, ROW), lambda i: (0, i, 0)),
            pl.BlockSpec((1, HID), lambda i: (0, 0)),
            pl.BlockSpec((BLK, 1), lambda i: (i, 0)),
        ],
        out_specs=[pl.BlockSpec((G, HID), lambda i: (0, 0))] * 3,
        out_shape=[out_t, out_t, out_t],
    )(lohi, acc2, b2r, batch_col)


def _final_body(osum, ocnt, omax, card, gen, wc1, bc1, wc2, bc2, wg, bg,
                wf1, bf1, wf2, bf2, out):
    gmean = osum[...] / jnp.maximum(ocnt[...][:, 0:1], 1.0)
    xg = jnp.concatenate([gmean, omax[...]], axis=1)
    c = jnp.maximum(jnp.dot(card[...], wc1[...],
                            preferred_element_type=f32) + bc1[...], 0.0)
    c = jnp.maximum(jnp.dot(c, wc2[...],
                            preferred_element_type=f32) + bc2[...], 0.0)
    g = jnp.maximum(jnp.dot(gen[...], wg[...],
                            preferred_element_type=f32) + bg[...], 0.0)
    comb = jnp.concatenate([xg, c, g], axis=1)
    o = jnp.maximum(jnp.dot(comb, wf1[...],
                            preferred_element_type=f32) + bf1[...], 0.0)
    out[...] = jnp.dot(o, wf2[...], preferred_element_type=f32) + bf2[...]


def _final(osum, ocnt, omax, card, gen, Wc1, bc1, Wc2, bc2, Wg, bg,
           Wf1, bf1, Wf2, bf2):
    return pl.pallas_call(
        _final_body,
        out_shape=jax.ShapeDtypeStruct((G, 30), f32),
    )(osum, ocnt, omax, card, gen, Wc1, bc1.reshape(1, -1), Wc2,
      bc2.reshape(1, -1), Wg, bg.reshape(1, -1), Wf1, bf1.reshape(1, -1),
      Wf2, bf2.reshape(1, -1))


def kernel(x, edge_index, batch, card, genome_feat, emb, W1, a_src1, a_dst1,
           b1, W2, a_src2, a_dst2, b2, Wc1, bc1, Wc2, bc2, Wg, bg, Wf1, bf1,
           Wf2, bf2):
    eye = jnp.eye(HEADS, dtype=f32)
    Asrc = (a_src1[:, :, None] * eye[:, None, :]).reshape(HID, HEADS)
    Adst = (a_dst1[:, :, None] * eye[:, None, :]).reshape(HID, HEADS)

    x_pad = jnp.concatenate(
        [x[:, 0].astype(i32), jnp.zeros((N_PAD - N,), i32)])
    loop = jnp.arange(N, dtype=i32)
    npad_e = E_PAD - E_TOT
    src_p = jnp.concatenate(
        [edge_index[0].astype(i32), loop, jnp.full((npad_e,), N, i32)])
    dst_p = jnp.concatenate(
        [edge_index[1].astype(i32), loop,
         (jnp.arange(npad_e, dtype=i32) % 32) * CHUNK])
    psrc = jnp.full((32 * CAP,), N, i32)
    pdst = jnp.repeat(jnp.arange(32, dtype=i32) * CHUNK, CAP)
    zrows = jnp.zeros((NPS, ROW), f32)
    batch_pad = jnp.concatenate(
        [batch.astype(i32), jnp.full((N_PAD - N,), G, i32)])
    lo = jnp.minimum(batch_pad[::BLK], G - 1)
    hi = jnp.minimum(batch_pad[BLK - 1::BLK], G - 1)
    lohi = jnp.stack([lo, hi], axis=1).astype(i32).reshape(-1, 1, 2)
    batch_col = batch_pad.reshape(N_PAD, 1)

    bsrc, bdst = _bucket_edges(src_p, dst_p, psrc, pdst)
    bsrc = bsrc.reshape(-1)
    bdst = bdst.reshape(-1)
    rk0, rk1, rk2, rk3, ark = _build_kmer_tables(emb, W1, Asrc, Adst)
    rec1, arn1 = _gather_nodes(x_pad, rk0, rk1, rk2, rk3, ark)
    rec1 = rec1.at[:, N, 32].set(-1e30)
    acc1 = _edge_pass(1, bsrc, bdst, rec1[0], rec1[1], rec1[2], rec1[3],
                      arn1, zrows)
    rec2, arn2 = _mid(acc1, W2, b1.reshape(1, HID), a_src2, a_dst2)
    rec2 = rec2.at[:, N, 32].set(-1e30)

    acc2 = _edge_pass(0, bsrc, bdst, rec2[0], rec2[1], rec2[2], rec2[3],
                      arn2, zrows)
    osum, ocnt, omax = _pool(acc2, b2.reshape(1, HID), batch_col, lohi)
    card_p = jnp.pad(card, ((0, 0), (0, 1)))
    Wc1_p = jnp.pad(Wc1, ((0, 1), (0, 0)))
    gen_p = jnp.pad(genome_feat, ((0, 0), (0, 5)))
    Wg_p = jnp.pad(Wg, ((0, 5), (0, 0)))
    return _final(osum, ocnt, omax, card_p, gen_p, Wc1_p, bc1, Wc2, bc2,
                  Wg_p, bg, Wf1, bf1, Wf2, bf2)
